# baseline (speedup 1.0000x reference)
import functools

import jax
import jax.numpy as jnp
from jax import lax
from jax.experimental import pallas as pl
from jax.experimental.pallas import tpu as pltpu
from jax.experimental.pallas import tpu_sc as plsc

_D = 128


@functools.lru_cache(maxsize=None)
def _embed_lookup(B: int, C: int):
    info = plsc.get_sparse_core_info()
    NC, NS = info.num_cores, info.num_subcores
    NW = NC * NS
    b_per_w = B // NW
    n_chunks = b_per_w // C
    assert b_per_w % C == 0 and B % NW == 0
    assert n_chunks >= 6 and n_chunks % 2 == 0
    mesh = plsc.VectorSubcoreMesh(core_axis_name="c", subcore_axis_name="s")

    V = 1024
    rows_per_tile = V // NS

    @functools.partial(
        pl.kernel,
        mesh=mesh,
        out_type=jax.ShapeDtypeStruct((B, _D), jnp.float32),
        scratch_types=[
            pltpu.VMEM((C,), jnp.int32),
            pltpu.VMEM((C,), jnp.int32),
            pltpu.VMEM((2, C, _D), jnp.float32),
            pltpu.VMEM_SHARED((V, _D), jnp.float32),
            pltpu.SemaphoreType.DMA,
            pltpu.SemaphoreType.DMA,
            pltpu.SemaphoreType.DMA,
            pltpu.SemaphoreType.DMA,
        ],
    )
    def k(idx_hbm, table_hbm, out_hbm, iv0, iv1, rows_v, table_s, g0, g1, s0, s1):
        sid = lax.axis_index("s")
        wid = sid * NC + lax.axis_index("c")
        base = wid * b_per_w
        gsem = (g0, g1)
        ssem = (s0, s1)
        idx_v = (iv0, iv1)

        r0 = sid * rows_per_tile
        pltpu.sync_copy(
            table_hbm.at[pl.ds(r0, rows_per_tile)],
            rows_v.at[0, pl.ds(0, rows_per_tile)],
        )
        pltpu.sync_copy(
            rows_v.at[0, pl.ds(0, rows_per_tile)],
            table_s.at[pl.ds(r0, rows_per_tile)],
        )
        plsc.subcore_barrier()

        def issue_gather(i, b):
            off = base + i * C
            pltpu.sync_copy(idx_hbm.at[pl.ds(off, C)], idx_v[b])
            pltpu.async_copy(table_s.at[idx_v[b]], rows_v.at[b], gsem[b])

        def issue_store(i, b):
            off = base + i * C
            pltpu.async_copy(rows_v.at[b], out_hbm.at[pl.ds(off, C)], ssem[b])

        def wait_gather(b):
            pltpu.make_async_copy(
                table_s.at[idx_v[b]], rows_v.at[b], gsem[b]
            ).wait()

        def wait_store(b):
            pltpu.make_async_copy(
                rows_v.at[b], out_hbm.at[pl.ds(base, C)], ssem[b]
            ).wait()

        issue_gather(0, 0)
        issue_gather(1, 1)

        def body(s, carry):
            for b in range(2):
                i = 2 + 2 * s + b
                wait_gather(b)
                issue_store(i - 2, b)
                wait_store(b)
                issue_gather(i, b)
            return carry

        lax.fori_loop(0, (n_chunks - 2) // 2, body, 0)

        wait_gather(0)
        issue_store(n_chunks - 2, 0)
        wait_gather(1)
        issue_store(n_chunks - 1, 1)
        wait_store(0)
        wait_store(1)

    return k


def kernel(atype, table):
    nf, nloc = atype.shape
    B = nf * nloc
    flat = atype.reshape(B)
    tpad = jnp.zeros((1024, _D), table.dtype).at[: table.shape[0]].set(table)
    out = _embed_lookup(B, 400)(flat, tpad)
    return out.reshape(nf, nloc, _D)


# speedup vs baseline: 1.0813x; 1.0813x over previous
import functools

import jax
import jax.numpy as jnp
from jax import lax
from jax.experimental import pallas as pl
from jax.experimental.pallas import tpu as pltpu
from jax.experimental.pallas import tpu_sc as plsc

_D = 128


@functools.lru_cache(maxsize=None)
def _embed_lookup(B: int, C: int):
    info = plsc.get_sparse_core_info()
    NC, NS = info.num_cores, info.num_subcores
    NW = NC * NS
    b_per_w = B // NW
    n_chunks = b_per_w // C
    assert b_per_w % C == 0 and B % NW == 0
    assert n_chunks >= 6 and n_chunks % 2 == 0
    mesh = plsc.VectorSubcoreMesh(core_axis_name="c", subcore_axis_name="s")

    V = 1024
    rows_per_tile = V // NS

    @functools.partial(
        pl.kernel,
        mesh=mesh,
        out_type=jax.ShapeDtypeStruct((B, _D), jnp.float32),
        scratch_types=[
            pltpu.VMEM((b_per_w,), jnp.int32),
            pltpu.VMEM((2, C, _D), jnp.float32),
            pltpu.VMEM_SHARED((V, _D), jnp.float32),
            pltpu.SemaphoreType.DMA,
            pltpu.SemaphoreType.DMA,
            pltpu.SemaphoreType.DMA,
            pltpu.SemaphoreType.DMA,
        ],
    )
    def k(idx_hbm, table_hbm, out_hbm, idx_v, rows_v, table_s, g0, g1, s0, s1):
        sid = lax.axis_index("s")
        wid = sid * NC + lax.axis_index("c")
        base = wid * b_per_w
        gsem = (g0, g1)
        ssem = (s0, s1)

        r0 = sid * rows_per_tile
        pltpu.sync_copy(
            table_hbm.at[pl.ds(r0, rows_per_tile)],
            rows_v.at[0, pl.ds(0, rows_per_tile)],
        )
        pltpu.sync_copy(
            rows_v.at[0, pl.ds(0, rows_per_tile)],
            table_s.at[pl.ds(r0, rows_per_tile)],
        )
        plsc.subcore_barrier()

        pltpu.sync_copy(idx_hbm.at[pl.ds(base, b_per_w)], idx_v)

        def issue_gather(i, b):
            pltpu.async_copy(
                table_s.at[idx_v.at[pl.ds(i * C, C)]], rows_v.at[b], gsem[b]
            )

        def issue_store(i, b):
            off = base + i * C
            pltpu.async_copy(rows_v.at[b], out_hbm.at[pl.ds(off, C)], ssem[b])

        def wait_gather(b):
            pltpu.make_async_copy(
                table_s.at[idx_v.at[pl.ds(0, C)]], rows_v.at[b], gsem[b]
            ).wait()

        def wait_store(b):
            pltpu.make_async_copy(
                rows_v.at[b], out_hbm.at[pl.ds(base, C)], ssem[b]
            ).wait()

        issue_gather(0, 0)
        issue_gather(1, 1)

        def body(s, carry):
            for b in range(2):
                i = 2 + 2 * s + b
                wait_gather(b)
                issue_store(i - 2, b)
                wait_store(b)
                issue_gather(i, b)
            return carry

        lax.fori_loop(0, (n_chunks - 2) // 2, body, 0)

        wait_gather(0)
        issue_store(n_chunks - 2, 0)
        wait_gather(1)
        issue_store(n_chunks - 1, 1)
        wait_store(0)
        wait_store(1)

    return k


def kernel(atype, table):
    nf, nloc = atype.shape
    B = nf * nloc
    flat = atype.reshape(B)
    tpad = jnp.zeros((1024, _D), table.dtype).at[: table.shape[0]].set(table)
    out = _embed_lookup(B, 320)(flat, tpad)
    return out.reshape(nf, nloc, _D)
